# baseline (speedup 1.0000x reference)
import jax
import jax.numpy as jnp
from jax.experimental import pallas as pl
from jax.experimental.pallas import tpu as pltpu

_BB = 40


def _body(nodes_ref, pooled_ref, mask_ref, locked_ref, out_ref, keep_ref):
    i = pl.program_id(0)

    @pl.when(i == 0)
    def _():
        lock = mask_ref[...] & locked_ref[...]
        keep_ref[pl.ds(0, mask_ref.shape[0]), :] = 1.0 - lock.astype(jnp.float32)

    off = pl.multiple_of(i * _BB, 8)
    keep = keep_ref[pl.ds(off, _BB), :][:, :, None]
    out_ref[...] = nodes_ref[...] + pooled_ref[...] * keep


def kernel(nodes, mask, pooled_edges, locked_nodes):
    B, N, D = nodes.shape
    nsteps = pl.cdiv(B, _BB)
    bs3 = pl.BlockSpec((_BB, N, D), lambda i: (i, 0, 0))
    bsm = pl.BlockSpec((B, N), lambda i: (0, 0))
    return pl.pallas_call(
        _body,
        grid=(nsteps,),
        in_specs=[bs3, bs3, bsm, bsm],
        out_specs=bs3,
        out_shape=jax.ShapeDtypeStruct((B, N, D), nodes.dtype),
        scratch_shapes=[pltpu.VMEM((nsteps * _BB, N), jnp.float32)],
    )(nodes, pooled_edges, mask, locked_nodes)


# speedup vs baseline: 1.0181x; 1.0181x over previous
import jax
import jax.numpy as jnp
from jax.experimental import pallas as pl
from jax.experimental.pallas import tpu as pltpu

_BB = 40


def _body(nodes_ref, pooled_ref, keep_ref, out_ref):
    keep = keep_ref[...][:, :, None]
    out_ref[...] = nodes_ref[...] + pooled_ref[...] * keep


def kernel(nodes, mask, pooled_edges, locked_nodes):
    B, N, D = nodes.shape
    keepf = 1.0 - (mask & locked_nodes).astype(jnp.float32)
    bs3 = pl.BlockSpec((_BB, N, D), lambda i: (i, 0, 0))
    bsm = pl.BlockSpec((_BB, N), lambda i: (i, 0))
    return pl.pallas_call(
        _body,
        grid=(pl.cdiv(B, _BB),),
        in_specs=[bs3, bs3, bsm],
        out_specs=bs3,
        out_shape=jax.ShapeDtypeStruct((B, N, D), nodes.dtype),
        compiler_params=pltpu.CompilerParams(
            dimension_semantics=("parallel",),
        ),
    )(nodes, pooled_edges, keepf)
